# baseline (speedup 1.0000x reference)
import functools

import jax
import jax.numpy as jnp
from jax import lax
from jax.experimental import pallas as pl
from jax.experimental.pallas import tpu as pltpu
from jax.experimental.pallas import tpu_sc as plsc

D_MODEL = 384
_SCALE = float(D_MODEL) ** 0.5
_LANES = 16

_NW = 32
_CHUNK = 32
_NBUF = 3


def _embed_body(idx_hbm, table_hbm, out_hbm, idx_v,
                g0, g1, g2, w0, w1, w2,
                gs0, gs1, gs2, ws0, ws1, ws2, *, n_chunks):
    gbufs, wbufs = (g0, g1, g2), (w0, w1, w2)
    gsems, wsems = (gs0, gs1, gs2), (ws0, ws1, ws2)
    wid = lax.axis_index("s") * 2 + lax.axis_index("c")
    base_row = wid * (n_chunks * _CHUNK)
    pltpu.sync_copy(idx_hbm.at[wid], idx_v)

    def gather_start(c, b):
        pltpu.make_async_copy(
            table_hbm.at[idx_v.at[c]], gbufs[b], gsems[b]).start()

    def gather_wait(b):
        pltpu.make_async_copy(table_hbm.at[idx_v.at[0]], gbufs[b],
                              gsems[b]).wait()

    def wb_start(c, b):
        pltpu.make_async_copy(
            wbufs[b], out_hbm.at[pl.ds(base_row + c * _CHUNK, _CHUNK)],
            wsems[b]).start()

    def wb_wait(b):
        pltpu.make_async_copy(wbufs[b],
                              out_hbm.at[pl.ds(0, _CHUNK)], wsems[b]).wait()

    def scale_chunk(b):
        def row_body(j, rcarry, gbuf=gbufs[b], wbuf=wbufs[b]):
            for i in range(D_MODEL // _LANES):
                sl = pl.ds(i * _LANES, _LANES)
                wbuf[j, sl] = gbuf[j, sl] * _SCALE
            return rcarry

        lax.fori_loop(0, _CHUNK, row_body, 0)

    for b in range(_NBUF):
        gather_start(b, b)

    n_full = (n_chunks // _NBUF) * _NBUF

    def pass_body(p, carry):
        cc = p * _NBUF
        for b in range(_NBUF):
            c = cc + b
            gather_wait(b)

            @pl.when(c >= _NBUF)
            def _(b=b):
                wb_wait(b)

            scale_chunk(b)

            @pl.when(c + _NBUF < n_chunks)
            def _(c=c, b=b):
                gather_start(c + _NBUF, b)

            wb_start(c, b)
        return carry

    lax.fori_loop(0, n_full // _NBUF, pass_body, 0)

    for c in range(n_full, n_chunks):
        b = c % _NBUF
        gather_wait(b)
        wb_wait(b)
        scale_chunk(b)
        wb_start(c, b)

    for b in range(_NBUF):
        wb_wait(b)


def kernel(x, W):
    orig_shape = x.shape
    b_total = x.size
    assert b_total % (_NW * _CHUNK) == 0
    n_chunks = b_total // (_NW * _CHUNK)
    idx = x.reshape(_NW, n_chunks, _CHUNK).astype(jnp.int32)

    mesh = plsc.VectorSubcoreMesh(core_axis_name="c", subcore_axis_name="s")
    run = functools.partial(
        pl.kernel,
        mesh=mesh,
        out_type=jax.ShapeDtypeStruct((b_total, D_MODEL), jnp.float32),
        scratch_types=(
            [pltpu.VMEM((n_chunks, _CHUNK), jnp.int32)]
            + [pltpu.VMEM((_CHUNK, D_MODEL), jnp.float32)] * (2 * _NBUF)
            + [pltpu.SemaphoreType.DMA] * (2 * _NBUF)
        ),
    )(functools.partial(_embed_body, n_chunks=n_chunks))
    out = run(idx, W)
    return out.reshape(*orig_shape, D_MODEL)
